# Initial kernel scaffold; baseline (speedup 1.0000x reference)
import functools
import jax
import jax.numpy as jnp
from jax import lax
from jax.experimental import pallas as pl
from jax.experimental.pallas import tpu as pltpu

B, N, K, T = 64, 128, 32, 8
HID, SP, HD, HE, HL, HLO = 64, 64, 2, 32, 48, 32
L = 2
TAU = 2.0
E = N * K


def _body(beta_ref, x_ref, ped_ref, maskf_ref, self_ref, idx_ref, hist_ref,
          Wef_ref, Wet_ref, bemb_ref,
          Wm1a_ref, Wm1b_ref, wrow_ref, bm1_ref, Wm2_ref, bm2_ref,
          Wu1a_ref, Wu1b_ref, bu1_ref, Wu2_ref, bu2_ref,
          Whebd_ref, bhebd_ref, WxT_ref, WhT_ref, bg_ref, Wlo_ref, blo_ref,
          Wsp_ref, bsp_ref, Wc1a_ref, Wc1b_ref, Wc1c_ref, Wc1t_ref, bc1_ref,
          Wd_ref, bd_ref, out_ref):
    f32 = jnp.float32
    beta = beta_ref[0, 0]
    bvec = jnp.full((1, 1), beta, f32)
    sb = jnp.sin(bvec)
    cb = jnp.cos(bvec)

    ped = ped_ref[0]
    trow = (beta * Wet_ref[0:1, :] + sb * Wet_ref[1:2, :]
            + cb * Wet_ref[2:3, :] + bemb_ref[...])
    h = jnp.dot(ped, Wef_ref[...], preferred_element_type=f32) + trow

    px = ped[:, 0:1]
    py = ped[:, 1:2]
    n2 = px * px + py * py

    idx = idx_ref[0]
    iot = lax.broadcasted_iota(jnp.int32, (N, K, N), 2)
    oh = (idx[:, :, None] == iot).astype(f32).reshape(E, N)

    maskf = maskf_ref[0]
    cnt = jnp.sum(maskf, axis=1, keepdims=True)
    maske = maskf.reshape(E, 1)

    P = jnp.concatenate([px, py, n2], axis=1)
    GP = jnp.dot(oh, P, preferred_element_type=f32)
    pxe = jnp.broadcast_to(px.reshape(N, 1, 1), (N, K, 1)).reshape(E, 1)
    pye = jnp.broadcast_to(py.reshape(N, 1, 1), (N, K, 1)).reshape(E, 1)
    n2e = jnp.broadcast_to(n2.reshape(N, 1, 1), (N, K, 1)).reshape(E, 1)
    d2 = n2e - 2.0 * (pxe * GP[:, 0:1] + pye * GP[:, 1:2]) + GP[:, 2:3]

    for l in range(L):
        ai = jnp.dot(h, Wm1a_ref[l], preferred_element_type=f32) + bm1_ref[l]
        aj = jnp.dot(h, Wm1b_ref[l], preferred_element_type=f32)
        ga = jnp.dot(oh, aj, preferred_element_type=f32)
        aie = jnp.broadcast_to(ai.reshape(N, 1, HID), (N, K, HID)).reshape(E, HID)
        e = jnp.maximum(aie + ga + d2 * wrow_ref[l], 0.0) * maske
        msum = jnp.sum(e.reshape(N, K, HID), axis=1)
        agg = jnp.dot(msum, Wm2_ref[l], preferred_element_type=f32) + cnt * bm2_ref[l]
        upd = jnp.maximum(
            jnp.dot(h, Wu1a_ref[l], preferred_element_type=f32)
            + jnp.dot(agg, Wu1b_ref[l], preferred_element_type=f32)
            + bu1_ref[l], 0.0)
        h = h + jnp.dot(upd, Wu2_ref[l], preferred_element_type=f32) + bu2_ref[l]

    he = jnp.maximum(
        jnp.dot(hist_ref[0], Whebd_ref[...], preferred_element_type=f32)
        + bhebd_ref[...], 0.0)
    hs = jnp.zeros((N, HL), f32)
    cs = jnp.zeros((N, HL), f32)
    for tt in range(T):
        xt = he[:, HE * tt:HE * (tt + 1)]
        gates = (jnp.dot(xt, WxT_ref[...], preferred_element_type=f32)
                 + jnp.dot(hs, WhT_ref[...], preferred_element_type=f32)
                 + bg_ref[...])
        ig = jax.nn.sigmoid(gates[:, 0:HL])
        fg = jax.nn.sigmoid(gates[:, HL:2 * HL])
        gg = jnp.tanh(gates[:, 2 * HL:3 * HL])
        og = jax.nn.sigmoid(gates[:, 3 * HL:4 * HL])
        cs = fg * cs + ig * gg
        hs = og * jnp.tanh(cs)
    hist_out = jnp.dot(hs, Wlo_ref[...], preferred_element_type=f32) + blo_ref[...]

    spatial = jnp.maximum(
        jnp.dot(x_ref[0], Wsp_ref[...], preferred_element_type=f32)
        + bsp_ref[...], 0.0)
    sf = self_ref[0]
    ds = sf[:, 4:5]
    sx = sf[:, 0:1]
    sy = sf[:, 1:2]
    temp = jnp.sqrt(sx * sx + sy * sy)
    temp_ = jnp.where(temp == 0.0, temp + 0.1, temp)
    predx = (ds * sx / temp_ - sf[:, 2:3]) / TAU
    predy = (ds * sy / temp_ - sf[:, 3:4]) / TAU

    trow2 = (beta * Wc1t_ref[0:1, :] + sb * Wc1t_ref[1:2, :]
             + cb * Wc1t_ref[2:3, :] + bc1_ref[...])
    hcat = (jnp.dot(h, Wc1a_ref[...], preferred_element_type=f32)
            + jnp.dot(hist_out, Wc1b_ref[...], preferred_element_type=f32)
            + jnp.dot(spatial, Wc1c_ref[...], preferred_element_type=f32)
            + trow2)
    outv = (jnp.dot(jnp.maximum(hcat, 0.0), Wd_ref[...], preferred_element_type=f32)
            + bd_ref[...])
    out_ref[0] = outv + jnp.concatenate([predx, predy], axis=1)


@jax.jit
def _run(beta2, x, ped, maskf, selff, idx, hist2,
         Wef, Wet, bemb, Wm1a, Wm1b, wrow, bm1r, Wm2, bm2r,
         Wu1a, Wu1b, bu1r, Wu2, bu2r,
         Whebd, bhebd, WxT, WhT, bg, Wlo, blo,
         Wsp, bsp, Wc1a, Wc1b, Wc1c, Wc1t, bc1, Wd, bd):
    def bspec(shape):
        return pl.BlockSpec(shape, lambda b: (b,) + (0,) * (len(shape) - 1))

    def wspec(shape):
        return pl.BlockSpec(shape, lambda b: (0,) * len(shape))

    grid = (B,)
    in_specs = [
        pl.BlockSpec((1, 1), lambda b: (b, 0), memory_space=pltpu.SMEM),
        bspec((1, N, 2)),
        bspec((1, N, 6)),
        bspec((1, N, K)),
        bspec((1, N, 5)),
        bspec((1, N, K)),
        bspec((1, N, T * HD)),
        wspec((6, HID)), wspec((3, HID)), wspec((1, HID)),
        wspec((L, HID, HID)), wspec((L, HID, HID)), wspec((L, 1, HID)),
        wspec((L, 1, HID)), wspec((L, HID, HID)), wspec((L, 1, HID)),
        wspec((L, HID, HID)), wspec((L, HID, HID)), wspec((L, 1, HID)),
        wspec((L, HID, HID)), wspec((L, 1, HID)),
        wspec((T * HD, T * HE)), wspec((1, T * HE)),
        wspec((HE, 4 * HL)), wspec((HL, 4 * HL)), wspec((1, 4 * HL)),
        wspec((HL, HLO)), wspec((1, HLO)),
        wspec((2, SP)), wspec((1, SP)),
        wspec((HID, SP // 2)), wspec((HLO, SP // 2)), wspec((SP, SP // 2)),
        wspec((3, SP // 2)), wspec((1, SP // 2)),
        wspec((SP // 2, 2)), wspec((1, 2)),
    ]
    out = pl.pallas_call(
        _body,
        grid=grid,
        in_specs=in_specs,
        out_specs=bspec((1, N, 2)),
        out_shape=jax.ShapeDtypeStruct((B, N, 2), jnp.float32),
    )(beta2, x, ped, maskf, selff, idx, hist2,
      Wef, Wet, bemb, Wm1a, Wm1b, wrow, bm1r, Wm2, bm2r,
      Wu1a, Wu1b, bu1r, Wu2, bu2r,
      Whebd, bhebd, WxT, WhT, bg, Wlo, blo,
      Wsp, bsp, Wc1a, Wc1b, Wc1c, Wc1t, bc1, Wd, bd)
    return out


def kernel(x, beta, ped_features, neigh_ped_mask, self_features, near_ped_idx,
           hist_feature, nei_list, t, W_emb, b_emb, Wm1, bm1, Wm2, bm2,
           Wu1, bu1, Wu2, bu2, W_sp, b_sp, W_he, b_he, Wih, Whh, bih, bhh,
           W_lo, b_lo, W_c1, b_c1, W_d, b_d):
    f32 = jnp.float32
    beta2 = beta.reshape(B, 1).astype(f32)
    maskf = neigh_ped_mask.astype(f32)
    idx = near_ped_idx.astype(jnp.int32)
    hist2 = hist_feature.reshape(B, N, T * HD).astype(f32)

    Wef = W_emb[:6].astype(f32)
    Wet = W_emb[6:9].astype(f32)
    bemb = b_emb.reshape(1, HID).astype(f32)
    Wm1a = Wm1[:, :HID, :].astype(f32)
    Wm1b = Wm1[:, HID:2 * HID, :].astype(f32)
    wrow = Wm1[:, 2 * HID, :].reshape(L, 1, HID).astype(f32)
    bm1r = bm1.reshape(L, 1, HID).astype(f32)
    Wm2c = Wm2.astype(f32)
    bm2r = bm2.reshape(L, 1, HID).astype(f32)
    Wu1a = Wu1[:, :HID, :].astype(f32)
    Wu1b = Wu1[:, HID:, :].astype(f32)
    bu1r = bu1.reshape(L, 1, HID).astype(f32)
    Wu2c = Wu2.astype(f32)
    bu2r = bu2.reshape(L, 1, HID).astype(f32)
    Whebd = jnp.zeros((T * HD, T * HE), f32)
    for tt in range(T):
        Whebd = Whebd.at[tt * HD:(tt + 1) * HD, tt * HE:(tt + 1) * HE].set(
            W_he.astype(f32))
    bhebd = jnp.tile(b_he.astype(f32), T).reshape(1, T * HE)
    WxT = Wih.T.astype(f32)
    WhT = Whh.T.astype(f32)
    bg = (bih + bhh).reshape(1, 4 * HL).astype(f32)
    Wlo = W_lo.astype(f32)
    blo = b_lo.reshape(1, HLO).astype(f32)
    Wsp = W_sp.astype(f32)
    bsp = b_sp.reshape(1, SP).astype(f32)
    Wc1a = W_c1[0:HID].astype(f32)
    Wc1b = W_c1[HID:HID + HLO].astype(f32)
    Wc1c = W_c1[HID + HLO:HID + HLO + SP].astype(f32)
    Wc1t = W_c1[HID + HLO + SP:].astype(f32)
    bc1 = b_c1.reshape(1, SP // 2).astype(f32)
    Wd = W_d.astype(f32)
    bd = b_d.reshape(1, 2).astype(f32)

    return _run(beta2, x.astype(f32), ped_features.astype(f32), maskf,
                self_features.astype(f32), idx, hist2,
                Wef, Wet, bemb, Wm1a, Wm1b, wrow, bm1r, Wm2c, bm2r,
                Wu1a, Wu1b, bu1r, Wu2c, bu2r,
                Whebd, bhebd, WxT, WhT, bg, Wlo, blo,
                Wsp, bsp, Wc1a, Wc1b, Wc1c, Wc1t, bc1, Wd, bd)


# speedup vs baseline: 14.1459x; 14.1459x over previous
import functools
import jax
import jax.numpy as jnp
from jax import lax
from jax.experimental import pallas as pl
from jax.experimental.pallas import tpu as pltpu

B, N, K, T = 64, 128, 32, 8
HID, SP, HD, HE, HL, HLO = 64, 64, 2, 32, 48, 32
L = 2
TAU = 2.0
E = N * K


def _body(beta_ref, x_ref, ped_ref, maskf_ref, maske_ref, self_ref, idx_ref, hist_ref,
          Wef_ref, Wet_ref, bemb_ref,
          Wm1a_ref, Wm1b_ref, wrow_ref, bm1_ref, Wm2_ref, bm2_ref,
          Wu1a_ref, Wu1b_ref, bu1_ref, Wu2_ref, bu2_ref,
          Whebd_ref, bhebd_ref, WxT_ref, WhT_ref, bg_ref, Wlo_ref, blo_ref,
          Wsp_ref, bsp_ref, Wc1a_ref, Wc1b_ref, Wc1c_ref, Wc1t_ref, bc1_ref,
          Wd_ref, bd_ref, out_ref):
    f32 = jnp.float32
    beta = beta_ref[0, 0, 0]
    bvec = jnp.full((1, 1), beta, f32)
    sb = jnp.sin(bvec)
    cb = jnp.cos(bvec)

    ped = ped_ref[0]
    trow = (beta * Wet_ref[0:1, :] + sb * Wet_ref[1:2, :]
            + cb * Wet_ref[2:3, :] + bemb_ref[...])
    h = jnp.dot(ped, Wef_ref[...], preferred_element_type=f32) + trow

    px = ped[:, 0:1]
    py = ped[:, 1:2]
    n2 = px * px + py * py

    idx = idx_ref[0]
    iot = lax.broadcasted_iota(jnp.int32, (N, K, N), 2)
    oh = (idx[:, :, None] == iot).astype(f32).reshape(E, N)

    maskf = maskf_ref[0]
    cnt = jnp.sum(maskf, axis=1, keepdims=True)
    maske = maske_ref[0]

    P = jnp.concatenate([px, py, n2], axis=1)
    GP = jnp.dot(oh, P, preferred_element_type=f32)
    pxe = jnp.broadcast_to(px.reshape(N, 1, 1), (N, K, 1)).reshape(E, 1)
    pye = jnp.broadcast_to(py.reshape(N, 1, 1), (N, K, 1)).reshape(E, 1)
    n2e = jnp.broadcast_to(n2.reshape(N, 1, 1), (N, K, 1)).reshape(E, 1)
    d2 = n2e - 2.0 * (pxe * GP[:, 0:1] + pye * GP[:, 1:2]) + GP[:, 2:3]

    for l in range(L):
        ai = jnp.dot(h, Wm1a_ref[l], preferred_element_type=f32) + bm1_ref[l]
        aj = jnp.dot(h, Wm1b_ref[l], preferred_element_type=f32)
        ga = jnp.dot(oh, aj, preferred_element_type=f32)
        aie = jnp.broadcast_to(ai.reshape(N, 1, HID), (N, K, HID)).reshape(E, HID)
        e = jnp.maximum(aie + ga + d2 * wrow_ref[l], 0.0) * maske
        msum = jnp.sum(e.reshape(N, K, HID), axis=1)
        agg = jnp.dot(msum, Wm2_ref[l], preferred_element_type=f32) + cnt * bm2_ref[l]
        upd = jnp.maximum(
            jnp.dot(h, Wu1a_ref[l], preferred_element_type=f32)
            + jnp.dot(agg, Wu1b_ref[l], preferred_element_type=f32)
            + bu1_ref[l], 0.0)
        h = h + jnp.dot(upd, Wu2_ref[l], preferred_element_type=f32) + bu2_ref[l]

    he = jnp.maximum(
        jnp.dot(hist_ref[0], Whebd_ref[...], preferred_element_type=f32)
        + bhebd_ref[...], 0.0)
    hs = jnp.zeros((N, HL), f32)
    cs = jnp.zeros((N, HL), f32)
    for tt in range(T):
        xt = he[:, HE * tt:HE * (tt + 1)]
        gates = (jnp.dot(xt, WxT_ref[...], preferred_element_type=f32)
                 + jnp.dot(hs, WhT_ref[...], preferred_element_type=f32)
                 + bg_ref[...])
        ig = jax.nn.sigmoid(gates[:, 0:HL])
        fg = jax.nn.sigmoid(gates[:, HL:2 * HL])
        gg = jnp.tanh(gates[:, 2 * HL:3 * HL])
        og = jax.nn.sigmoid(gates[:, 3 * HL:4 * HL])
        cs = fg * cs + ig * gg
        hs = og * jnp.tanh(cs)
    hist_out = jnp.dot(hs, Wlo_ref[...], preferred_element_type=f32) + blo_ref[...]

    spatial = jnp.maximum(
        jnp.dot(x_ref[0], Wsp_ref[...], preferred_element_type=f32)
        + bsp_ref[...], 0.0)
    sf = self_ref[0]
    ds = sf[:, 4:5]
    sx = sf[:, 0:1]
    sy = sf[:, 1:2]
    temp = jnp.sqrt(sx * sx + sy * sy)
    temp_ = jnp.where(temp == 0.0, temp + 0.1, temp)
    predx = (ds * sx / temp_ - sf[:, 2:3]) / TAU
    predy = (ds * sy / temp_ - sf[:, 3:4]) / TAU

    trow2 = (beta * Wc1t_ref[0:1, :] + sb * Wc1t_ref[1:2, :]
             + cb * Wc1t_ref[2:3, :] + bc1_ref[...])
    hcat = (jnp.dot(h, Wc1a_ref[...], preferred_element_type=f32)
            + jnp.dot(hist_out, Wc1b_ref[...], preferred_element_type=f32)
            + jnp.dot(spatial, Wc1c_ref[...], preferred_element_type=f32)
            + trow2)
    outv = (jnp.dot(jnp.maximum(hcat, 0.0), Wd_ref[...], preferred_element_type=f32)
            + bd_ref[...])
    out_ref[0] = outv + jnp.concatenate([predx, predy], axis=1)


@jax.jit
def _run(beta2, x, ped, maskf, maske3, selff, idx, hist2,
         Wef, Wet, bemb, Wm1a, Wm1b, wrow, bm1r, Wm2, bm2r,
         Wu1a, Wu1b, bu1r, Wu2, bu2r,
         Whebd, bhebd, WxT, WhT, bg, Wlo, blo,
         Wsp, bsp, Wc1a, Wc1b, Wc1c, Wc1t, bc1, Wd, bd):
    def bspec(shape):
        return pl.BlockSpec(shape, lambda b: (b,) + (0,) * (len(shape) - 1))

    def wspec(shape):
        return pl.BlockSpec(shape, lambda b: (0,) * len(shape))

    grid = (B,)
    in_specs = [
        pl.BlockSpec((1, 1, 1), lambda b: (b, 0, 0), memory_space=pltpu.SMEM),
        bspec((1, N, 2)),
        bspec((1, N, 6)),
        bspec((1, N, K)),
        bspec((1, E, 1)),
        bspec((1, N, 5)),
        bspec((1, N, K)),
        bspec((1, N, T * HD)),
        wspec((6, HID)), wspec((3, HID)), wspec((1, HID)),
        wspec((L, HID, HID)), wspec((L, HID, HID)), wspec((L, 1, HID)),
        wspec((L, 1, HID)), wspec((L, HID, HID)), wspec((L, 1, HID)),
        wspec((L, HID, HID)), wspec((L, HID, HID)), wspec((L, 1, HID)),
        wspec((L, HID, HID)), wspec((L, 1, HID)),
        wspec((T * HD, T * HE)), wspec((1, T * HE)),
        wspec((HE, 4 * HL)), wspec((HL, 4 * HL)), wspec((1, 4 * HL)),
        wspec((HL, HLO)), wspec((1, HLO)),
        wspec((2, SP)), wspec((1, SP)),
        wspec((HID, SP // 2)), wspec((HLO, SP // 2)), wspec((SP, SP // 2)),
        wspec((3, SP // 2)), wspec((1, SP // 2)),
        wspec((SP // 2, 2)), wspec((1, 2)),
    ]
    out = pl.pallas_call(
        _body,
        grid=grid,
        in_specs=in_specs,
        out_specs=bspec((1, N, 2)),
        out_shape=jax.ShapeDtypeStruct((B, N, 2), jnp.float32),
    )(beta2, x, ped, maskf, maske3, selff, idx, hist2,
      Wef, Wet, bemb, Wm1a, Wm1b, wrow, bm1r, Wm2, bm2r,
      Wu1a, Wu1b, bu1r, Wu2, bu2r,
      Whebd, bhebd, WxT, WhT, bg, Wlo, blo,
      Wsp, bsp, Wc1a, Wc1b, Wc1c, Wc1t, bc1, Wd, bd)
    return out


def kernel(x, beta, ped_features, neigh_ped_mask, self_features, near_ped_idx,
           hist_feature, nei_list, t, W_emb, b_emb, Wm1, bm1, Wm2, bm2,
           Wu1, bu1, Wu2, bu2, W_sp, b_sp, W_he, b_he, Wih, Whh, bih, bhh,
           W_lo, b_lo, W_c1, b_c1, W_d, b_d):
    f32 = jnp.float32
    beta2 = beta.reshape(B, 1, 1).astype(f32)
    maskf = neigh_ped_mask.astype(f32)
    maske3 = maskf.reshape(B, E, 1)
    idx = near_ped_idx.astype(jnp.int32)
    hist2 = hist_feature.reshape(B, N, T * HD).astype(f32)

    Wef = W_emb[:6].astype(f32)
    Wet = W_emb[6:9].astype(f32)
    bemb = b_emb.reshape(1, HID).astype(f32)
    Wm1a = Wm1[:, :HID, :].astype(f32)
    Wm1b = Wm1[:, HID:2 * HID, :].astype(f32)
    wrow = Wm1[:, 2 * HID, :].reshape(L, 1, HID).astype(f32)
    bm1r = bm1.reshape(L, 1, HID).astype(f32)
    Wm2c = Wm2.astype(f32)
    bm2r = bm2.reshape(L, 1, HID).astype(f32)
    Wu1a = Wu1[:, :HID, :].astype(f32)
    Wu1b = Wu1[:, HID:, :].astype(f32)
    bu1r = bu1.reshape(L, 1, HID).astype(f32)
    Wu2c = Wu2.astype(f32)
    bu2r = bu2.reshape(L, 1, HID).astype(f32)
    Whebd = jnp.zeros((T * HD, T * HE), f32)
    for tt in range(T):
        Whebd = Whebd.at[tt * HD:(tt + 1) * HD, tt * HE:(tt + 1) * HE].set(
            W_he.astype(f32))
    bhebd = jnp.tile(b_he.astype(f32), T).reshape(1, T * HE)
    WxT = Wih.T.astype(f32)
    WhT = Whh.T.astype(f32)
    bg = (bih + bhh).reshape(1, 4 * HL).astype(f32)
    Wlo = W_lo.astype(f32)
    blo = b_lo.reshape(1, HLO).astype(f32)
    Wsp = W_sp.astype(f32)
    bsp = b_sp.reshape(1, SP).astype(f32)
    Wc1a = W_c1[0:HID].astype(f32)
    Wc1b = W_c1[HID:HID + HLO].astype(f32)
    Wc1c = W_c1[HID + HLO:HID + HLO + SP].astype(f32)
    Wc1t = W_c1[HID + HLO + SP:].astype(f32)
    bc1 = b_c1.reshape(1, SP // 2).astype(f32)
    Wd = W_d.astype(f32)
    bd = b_d.reshape(1, 2).astype(f32)

    return _run(beta2, x.astype(f32), ped_features.astype(f32), maskf, maske3,
                self_features.astype(f32), idx, hist2,
                Wef, Wet, bemb, Wm1a, Wm1b, wrow, bm1r, Wm2c, bm2r,
                Wu1a, Wu1b, bu1r, Wu2c, bu2r,
                Whebd, bhebd, WxT, WhT, bg, Wlo, blo,
                Wsp, bsp, Wc1a, Wc1b, Wc1c, Wc1t, bc1, Wd, bd)
